# baseline (speedup 1.0000x reference)
import functools

import jax
import jax.numpy as jnp
from jax import lax
from jax.experimental import pallas as pl
from jax.experimental.pallas import tpu as pltpu
from jax.experimental.pallas import tpu_sc as plsc

FEATURE_NUM = 1040000
LATENT_DIM = 16
FIELD_NUM = 26
BATCH = 16384

NC = 2
NS = 16
NW = NC * NS
BW = BATCH // NW
NT = BATCH // 128
TW = NT // NW
RT = FEATURE_NUM // 128
GSTRIDE = RT * 1024

_mesh = plsc.VectorSubcoreMesh(core_axis_name="c", subcore_axis_name="s")


@functools.partial(
    pl.kernel,
    out_type=jax.ShapeDtypeStruct(
        (FIELD_NUM, LATENT_DIM // 8, NT, 1024), jnp.float32
    ),
    mesh=_mesh,
    compiler_params=pltpu.CompilerParams(
        needs_layout_passes=False, use_tc_tiling_on_sc=False
    ),
    scratch_types=[
        pltpu.VMEM((2, BW), jnp.int32),
        pltpu.VMEM((BW,), jnp.int32),
        pltpu.VMEM((2, TW, LATENT_DIM * 128), jnp.int32),
        pltpu.VMEM((2, 2, TW, 1024), jnp.float32),
        pltpu.VMEM((FIELD_NUM, 16), jnp.float32),
        pltpu.VMEM((16 * 128,), jnp.float32),
        pltpu.SemaphoreType.DMA,
        pltpu.SemaphoreType.DMA,
        pltpu.SemaphoreType.DMA,
        pltpu.SemaphoreType.DMA,
        pltpu.SemaphoreType.DMA,
        pltpu.SemaphoreType.DMA,
        pltpu.SemaphoreType.DMA,
        pltpu.SemaphoreType.DMA,
    ],
)
def _sc_embed(xt_hbm, thr_hbm, tabf_hbm, out_hbm, idx2_v, base_v, addr_v,
              trans2_v, thr_v, drain_v, sem0, sem1, sem2, sem3, semi0, semi1,
              semo0, semo1):
    wid = lax.axis_index("s") * NC + lax.axis_index("c")
    b0 = wid * BW
    t0 = wid * TW
    sems = (sem0, sem1, sem2, sem3)
    semi = (semi0, semi1)
    semo = (semo0, semo1)

    pltpu.sync_copy(thr_hbm, thr_v)
    pltpu.sync_copy(xt_hbm.at[0, pl.ds(b0, BW)], idx2_v.at[0])

    def build_and_fire(f, p, tl):
        def d_body(d, dcarry):
            off = (
                lax.shift_right_logical(d, 3) * GSTRIDE
                + lax.bitwise_and(d, jnp.int32(7)) * 128
            )
            for q in range(8):
                addr_v[p, tl, pl.ds(d * 128 + 16 * q, 16)] = (
                    base_v[pl.ds(tl * 128 + 16 * q, 16)] + off
                )
            return dcarry

        lax.fori_loop(0, LATENT_DIM, d_body, 0)
        for g in range(2):
            pltpu.async_copy(
                tabf_hbm.at[addr_v.at[p, tl, pl.ds(g * 1024, 1024)]],
                trans2_v.at[p, g, tl],
                sems[tl],
            )

    def drain(tl):
        pltpu.make_async_copy(
            tabf_hbm.at[pl.ds(0, 16 * 128)], drain_v, sems[tl]
        ).wait()

    def compute(f, p, tl):
        t_vec = thr_v[f, :]
        zeros = (jnp.zeros((16,), jnp.float32),) * 8

        def sum_body(d, sums):
            return tuple(
                sums[q]
                + jnp.abs(
                    trans2_v[
                        p,
                        lax.shift_right_logical(d, 3),
                        tl,
                        pl.ds(lax.bitwise_and(d, jnp.int32(7)) * 128 + 16 * q, 16),
                    ]
                )
                for q in range(8)
            )

        sums = lax.fori_loop(0, LATENT_DIM, sum_body, zeros)
        masks = tuple(
            ((sums[q] - t_vec) > 0).astype(jnp.float32) for q in range(8)
        )

        def apply_body(d, dcarry):
            g = lax.shift_right_logical(d, 3)
            dd = lax.bitwise_and(d, jnp.int32(7))
            for q in range(8):
                trans2_v[p, g, tl, pl.ds(dd * 128 + 16 * q, 16)] = (
                    trans2_v[p, g, tl, pl.ds(dd * 128 + 16 * q, 16)] * masks[q]
                )
            return dcarry

        lax.fori_loop(0, LATENT_DIM, apply_body, 0)

    def prelude(f, p, first):
        if not first:
            @pl.when(f >= 2)
            def _():
                for g in range(2):
                    pltpu.make_async_copy(
                        trans2_v.at[p, g], out_hbm.at[0, g, pl.ds(t0, TW)],
                        semo[p],
                    ).wait()

        def _prefetch():
            pltpu.async_copy(
                xt_hbm.at[f + 1, pl.ds(b0, BW)], idx2_v.at[1 - p],
                semi[1 - p],
            )

        if first:
            _prefetch()
        else:
            pl.when(f + 1 < FIELD_NUM)(_prefetch)

        if not first:
            pltpu.make_async_copy(
                xt_hbm.at[0, pl.ds(b0, BW)], idx2_v.at[p], semi[p]
            ).wait()

        for q in range(BW // 16):
            v = idx2_v[p, pl.ds(16 * q, 16)]
            base_v[pl.ds(16 * q, 16)] = (
                lax.shift_left(lax.shift_right_logical(v, 7), 10)
                + lax.bitwise_and(v, jnp.int32(127))
            )

        build_and_fire(f, p, 0)
        build_and_fire(f, p, 1)

    def body(f2, p):
        f = f2 * 2 + p
        build_and_fire(f, p, 2)
        drain(0)
        compute(f, p, 0)
        build_and_fire(f, p, 3)
        drain(1)
        compute(f, p, 1)
        if p == 0:
            prelude(f + 1, 1, False)
        else:
            @pl.when(f2 + 1 < FIELD_NUM // 2)
            def _():
                prelude(f + 1, 0, False)
        drain(2)
        compute(f, p, 2)
        drain(3)
        compute(f, p, 3)
        for g in range(2):
            pltpu.async_copy(
                trans2_v.at[p, g], out_hbm.at[f, g, pl.ds(t0, TW)], semo[p]
            )

    prelude(0, 0, True)

    def pair_body(f2, carry):
        body(f2, 0)
        body(f2, 1)
        return carry

    lax.fori_loop(0, FIELD_NUM // 2, pair_body, 0)

    for p in range(2):
        for g in range(2):
            pltpu.make_async_copy(
                trans2_v.at[p, g], out_hbm.at[0, g, pl.ds(t0, TW)], semo[p]
            ).wait()


@jax.jit
def kernel(x, phase, embedding, threshold):
    xt = x.T
    thr = jnp.broadcast_to(threshold, (FIELD_NUM, 16))
    tabf = (
        embedding.T.reshape(2, 8, RT, 128).transpose(0, 2, 1, 3).reshape(-1)
    )
    out4 = _sc_embed(xt, thr, tabf)
    out5 = out4.reshape(FIELD_NUM, LATENT_DIM // 8, NT, 8, 128)
    return out5.transpose(2, 4, 0, 1, 3).reshape(BATCH, FIELD_NUM, LATENT_DIM)


# speedup vs baseline: 1.0076x; 1.0076x over previous
import functools

import jax
import jax.numpy as jnp
from jax import lax
from jax.experimental import pallas as pl
from jax.experimental.pallas import tpu as pltpu
from jax.experimental.pallas import tpu_sc as plsc

FEATURE_NUM = 1040000
LATENT_DIM = 16
FIELD_NUM = 26
BATCH = 16384

NC = 2
NS = 16
NW = NC * NS
BW = BATCH // NW
NT = BATCH // 128
TW = NT // NW
RT = FEATURE_NUM // 128
GSTRIDE = RT * 1024

_mesh = plsc.VectorSubcoreMesh(core_axis_name="c", subcore_axis_name="s")


@functools.partial(
    pl.kernel,
    out_type=jax.ShapeDtypeStruct(
        (FIELD_NUM, LATENT_DIM // 8, NT, 8, 128), jnp.float32
    ),
    mesh=_mesh,
    compiler_params=pltpu.CompilerParams(
        needs_layout_passes=False, use_tc_tiling_on_sc=False
    ),
    scratch_types=[
        pltpu.VMEM((2, BW), jnp.int32),
        pltpu.VMEM((BW,), jnp.int32),
        pltpu.VMEM((2, TW, LATENT_DIM, 128), jnp.int32),
        pltpu.VMEM((2, 2, TW, 8, 128), jnp.float32),
        pltpu.VMEM((FIELD_NUM, 16), jnp.float32),
        pltpu.VMEM((16 * 128,), jnp.float32),
        pltpu.SemaphoreType.DMA,
        pltpu.SemaphoreType.DMA,
        pltpu.SemaphoreType.DMA,
        pltpu.SemaphoreType.DMA,
        pltpu.SemaphoreType.DMA,
        pltpu.SemaphoreType.DMA,
        pltpu.SemaphoreType.DMA,
        pltpu.SemaphoreType.DMA,
    ],
)
def _sc_embed(xt_hbm, thr_hbm, tabf_hbm, out_hbm, idx2_v, base_v, addr_v,
              trans2_v, thr_v, drain_v, sem0, sem1, sem2, sem3, semi0, semi1,
              semo0, semo1):
    wid = lax.axis_index("s") * NC + lax.axis_index("c")
    b0 = wid * BW
    t0 = wid * TW
    sems = (sem0, sem1, sem2, sem3)
    semi = (semi0, semi1)
    semo = (semo0, semo1)

    pltpu.sync_copy(thr_hbm, thr_v)
    pltpu.sync_copy(xt_hbm.at[0, pl.ds(b0, BW)], idx2_v.at[0])

    def build_and_fire(f, p, tl):
        def d_body(d, dcarry):
            off = (
                lax.shift_right_logical(d, 3) * GSTRIDE
                + lax.bitwise_and(d, jnp.int32(7)) * 128
            )
            for q in range(8):
                addr_v[p, tl, d, pl.ds(16 * q, 16)] = (
                    base_v[pl.ds(tl * 128 + 16 * q, 16)] + off
                )
            return dcarry

        lax.fori_loop(0, LATENT_DIM, d_body, 0)
        for g in range(2):
            for dd in range(8):
                pltpu.async_copy(
                    tabf_hbm.at[addr_v.at[p, tl, 8 * g + dd]],
                    trans2_v.at[p, g, tl, dd],
                    sems[tl],
                )

    def drain(tl):
        pltpu.make_async_copy(
            tabf_hbm.at[pl.ds(0, 16 * 128)], drain_v, sems[tl]
        ).wait()

    def compute(f, p, tl):
        t_vec = thr_v[f, :]
        zeros = (jnp.zeros((16,), jnp.float32),) * 8

        def sum_body(d, sums):
            return tuple(
                sums[q]
                + jnp.abs(
                    trans2_v[
                        p,
                        lax.shift_right_logical(d, 3),
                        tl,
                        lax.bitwise_and(d, jnp.int32(7)),
                        pl.ds(16 * q, 16),
                    ]
                )
                for q in range(8)
            )

        sums = lax.fori_loop(0, LATENT_DIM, sum_body, zeros)
        masks = tuple(
            ((sums[q] - t_vec) > 0).astype(jnp.float32) for q in range(8)
        )

        def apply_body(d, dcarry):
            g = lax.shift_right_logical(d, 3)
            dd = lax.bitwise_and(d, jnp.int32(7))
            for q in range(8):
                trans2_v[p, g, tl, dd, pl.ds(16 * q, 16)] = (
                    trans2_v[p, g, tl, dd, pl.ds(16 * q, 16)] * masks[q]
                )
            return dcarry

        lax.fori_loop(0, LATENT_DIM, apply_body, 0)

    def prelude(f, p, first):
        if not first:
            @pl.when(f >= 2)
            def _():
                for g in range(2):
                    pltpu.make_async_copy(
                        trans2_v.at[p, g], out_hbm.at[0, g, pl.ds(t0, TW)],
                        semo[p],
                    ).wait()

        def _prefetch():
            pltpu.async_copy(
                xt_hbm.at[f + 1, pl.ds(b0, BW)], idx2_v.at[1 - p],
                semi[1 - p],
            )

        if first:
            _prefetch()
        else:
            pl.when(f + 1 < FIELD_NUM)(_prefetch)

        if not first:
            pltpu.make_async_copy(
                xt_hbm.at[0, pl.ds(b0, BW)], idx2_v.at[p], semi[p]
            ).wait()

        for q in range(BW // 16):
            v = idx2_v[p, pl.ds(16 * q, 16)]
            base_v[pl.ds(16 * q, 16)] = (
                lax.shift_left(lax.shift_right_logical(v, 7), 10)
                + lax.bitwise_and(v, jnp.int32(127))
            )

        build_and_fire(f, p, 0)
        build_and_fire(f, p, 1)

    def body(f2, p):
        f = f2 * 2 + p
        build_and_fire(f, p, 2)
        drain(0)
        compute(f, p, 0)
        build_and_fire(f, p, 3)
        drain(1)
        compute(f, p, 1)
        if p == 0:
            prelude(f + 1, 1, False)
        else:
            @pl.when(f2 + 1 < FIELD_NUM // 2)
            def _():
                prelude(f + 1, 0, False)
        drain(2)
        compute(f, p, 2)
        drain(3)
        compute(f, p, 3)
        for g in range(2):
            pltpu.async_copy(
                trans2_v.at[p, g], out_hbm.at[f, g, pl.ds(t0, TW)], semo[p]
            )

    prelude(0, 0, True)

    def pair_body(f2, carry):
        body(f2, 0)
        body(f2, 1)
        return carry

    lax.fori_loop(0, FIELD_NUM // 2, pair_body, 0)

    for p in range(2):
        for g in range(2):
            pltpu.make_async_copy(
                trans2_v.at[p, g], out_hbm.at[0, g, pl.ds(t0, TW)], semo[p]
            ).wait()


@jax.jit
def kernel(x, phase, embedding, threshold):
    xt = x.T
    thr = jnp.broadcast_to(threshold, (FIELD_NUM, 16))
    tabf = (
        embedding.T.reshape(2, 8, RT, 128).transpose(0, 2, 1, 3).reshape(-1)
    )
    out5 = _sc_embed(xt, thr, tabf)
    return out5.transpose(2, 4, 0, 1, 3).reshape(BATCH, FIELD_NUM, LATENT_DIM)
